# baseline (speedup 1.0000x reference)
import functools

import jax
import jax.numpy as jnp
from jax import lax
from jax.experimental import pallas as pl
from jax.experimental.pallas import tpu as pltpu
from jax.experimental.pallas import tpu_sc as plsc

_NUM_MATERIALS = 1000000
_EMBED_DIM = 64
_BATCH = 16384

_NC = 2
_NS = 16
_NW = _NC * _NS
_B_PER_W = _BATCH // _NW
_GRP = 16
_NGRP = _B_PER_W // _GRP
_NPH = 4
_BPP = 8 // _NPH
_NSTEP = _NGRP * _NPH

_mesh = plsc.VectorSubcoreMesh(core_axis_name="c", subcore_axis_name="s")


@functools.partial(
    pl.kernel,
    mesh=_mesh,
    out_type=jax.ShapeDtypeStruct((_EMBED_DIM, _BATCH), jnp.float32),
    scratch_types=[
        pltpu.VMEM((_B_PER_W,), jnp.int32),
        pltpu.VMEM((2, _GRP, _BPP, 8, 128), jnp.float32),
        pltpu.VMEM((_EMBED_DIM, _B_PER_W), jnp.float32),
        pltpu.SemaphoreType.DMA,
    ],
    compiler_params=pltpu.CompilerParams(needs_layout_passes=False),
)
def _gather_kernel(table_hbm, idx_hbm, out_hbm, idx_v, buf_v, cols_v, sem):
    wid = lax.axis_index("s") * _NC + lax.axis_index("c")
    base = wid * _B_PER_W

    pltpu.sync_copy(idx_hbm.at[pl.ds(base, _B_PER_W)], idx_v)
    lane = lax.iota(jnp.int32, 16)

    def step_copies(s):
        w = lax.shift_right_logical(s, 2)
        ph = s & (_NPH - 1)
        par = s & 1
        rvec = idx_v[pl.ds(pl.multiple_of(w * _GRP, _GRP), _GRP)]
        return [
            pltpu.make_async_copy(
                table_hbm.at[
                    pl.ds(ph * _BPP, _BPP),
                    :,
                    pl.ds(pl.multiple_of(rvec[l] & -128, 128), 128),
                ],
                buf_v.at[par, l],
                sem,
            )
            for l in range(_GRP)
        ]

    def fire(s):
        for cp in step_copies(s):
            cp.start()

    fire(0)

    def step_body(s, carry):
        @pl.when(s < _NSTEP - 1)
        def _():
            fire(s + 1)

        par_w = s & 1
        for l in range(_GRP):
            pltpu.make_async_copy(
                table_hbm.at[pl.ds(0, _BPP), :, pl.ds(0, 128)],
                buf_v.at[par_w, l],
                sem,
            ).wait()

        w = lax.shift_right_logical(s, 2)
        ph = s & (_NPH - 1)
        par = s & 1
        goff = pl.multiple_of(w * _GRP, _GRP)
        rm_vec = idx_v[pl.ds(goff, _GRP)] & 127
        par_vec = jnp.full((16,), 0, jnp.int32) + par
        for cc in range(_BPP * 8):
            vals = plsc.load_gather(
                buf_v,
                [
                    par_vec,
                    lane,
                    jnp.full((16,), cc // 8, jnp.int32),
                    jnp.full((16,), cc % 8, jnp.int32),
                    rm_vec,
                ],
            )
            cols_v[ph * (_BPP * 8) + cc, pl.ds(goff, _GRP)] = vals
        return carry

    lax.fori_loop(0, _NSTEP, step_body, 0)
    pltpu.sync_copy(cols_v, out_hbm.at[:, pl.ds(base, _B_PER_W)])


def kernel(embeddings, material_index):
    table3 = embeddings.T.reshape(8, 8, _NUM_MATERIALS)
    out_t = _gather_kernel(table3, material_index)
    return out_t.T


# speedup vs baseline: 1.0588x; 1.0588x over previous
import functools

import jax
import jax.numpy as jnp
from jax import lax
from jax.experimental import pallas as pl
from jax.experimental.pallas import tpu as pltpu
from jax.experimental.pallas import tpu_sc as plsc

_NUM_MATERIALS = 1000000
_EMBED_DIM = 64
_BATCH = 16384

_NC = 2
_NS = 16
_NW = _NC * _NS
_B_PER_W = _BATCH // _NW
_GRP = 16
_NGRP = _B_PER_W // _GRP
_NPH = 2
_BPP = 8 // _NPH

_mesh = plsc.VectorSubcoreMesh(core_axis_name="c", subcore_axis_name="s")


@functools.partial(
    pl.kernel,
    mesh=_mesh,
    out_type=jax.ShapeDtypeStruct((_BATCH, _EMBED_DIM), jnp.float32),
    scratch_types=[
        pltpu.VMEM((_B_PER_W,), jnp.int32),
        pltpu.VMEM((_B_PER_W,), jnp.int32),
        pltpu.VMEM((_GRP,), jnp.int32),
        pltpu.VMEM((_GRP, _BPP, 8, 128), jnp.float32),
        pltpu.VMEM((_GRP, _EMBED_DIM), jnp.float32),
        pltpu.SemaphoreType.DMA,
        pltpu.SemaphoreType.DMA,
    ],
    compiler_params=pltpu.CompilerParams(needs_layout_passes=False),
)
def _gather_kernel(table_hbm, idx_hbm, pos_hbm, out_hbm, idx_v, pos_v,
                   blk_v, buf_v, row_v, sem, sem_out):
    wid = lax.axis_index("s") * _NC + lax.axis_index("c")
    base = wid * _B_PER_W

    pltpu.sync_copy(idx_hbm.at[pl.ds(base, _B_PER_W)], idx_v)
    pltpu.sync_copy(pos_hbm.at[pl.ds(base, _B_PER_W)], pos_v)
    lane = lax.iota(jnp.int32, 16)
    lprev = jnp.maximum(lane - 1, 0)
    zeros = jnp.full((16,), 0, jnp.int32)

    def wave_body(w, carry):
        goff = pl.multiple_of(w * _GRP, _GRP)
        rvec = idx_v[pl.ds(goff, _GRP)]
        pvec = pos_v[pl.ds(goff, _GRP)]
        rm_vec = rvec & 127
        blk = lax.shift_right_logical(rvec, 7)
        blk_v[pl.ds(0, _GRP)] = blk
        blk_prev = plsc.load_gather(blk_v, [lprev])
        newi = jnp.where((blk != blk_prev) | (lane == 0), 1, 0).astype(
            jnp.int32
        )
        slot_vec = plsc.cumsum(newi) - 1

        for ph in range(_NPH):
            for l in range(_GRP):
                @pl.when(newi[l] > 0)
                def _(l=l, ph=ph):
                    pltpu.make_async_copy(
                        table_hbm.at[
                            pl.ds(ph * _BPP, _BPP),
                            :,
                            pl.ds(
                                pl.multiple_of(rvec[l] & -128, 128), 128
                            ),
                        ],
                        buf_v.at[slot_vec[l]],
                        sem,
                    ).start()

            if ph == 0:
                @pl.when(w > 0)
                def _():
                    for l in range(_GRP):
                        pltpu.make_async_copy(
                            row_v.at[l], out_hbm.at[0], sem_out
                        ).wait()

            for l in range(_GRP):
                @pl.when(newi[l] > 0)
                def _(l=l):
                    pltpu.make_async_copy(
                        table_hbm.at[pl.ds(0, _BPP), :, pl.ds(0, 128)],
                        buf_v.at[0],
                        sem,
                    ).wait()

            for half in range(2):
                for l in range(_GRP):
                    vals = plsc.load_gather(
                        buf_v,
                        [
                            zeros + slot_vec[l],
                            half * 2 + (lane >> 3),
                            lane & 7,
                            zeros + rm_vec[l],
                        ],
                    )
                    row_v[l, pl.ds(ph * 32 + half * 16, 16)] = vals

        for l in range(_GRP):
            pltpu.make_async_copy(
                row_v.at[l], out_hbm.at[pvec[l]], sem_out
            ).start()
        return carry

    lax.fori_loop(0, _NGRP, wave_body, 0)
    for l in range(_GRP):
        pltpu.make_async_copy(row_v.at[l], out_hbm.at[0], sem_out).wait()


def kernel(embeddings, material_index):
    table3 = embeddings.T.reshape(8, 8, _NUM_MATERIALS)
    sidx, spos = lax.sort_key_val(
        material_index, lax.iota(jnp.int32, _BATCH)
    )
    return _gather_kernel(table3, sidx, spos)
